# baseline (speedup 1.0000x reference)
import functools

import jax
import jax.numpy as jnp
from jax import lax
from jax.experimental import pallas as pl
from jax.experimental.pallas import tpu as pltpu
from jax.experimental.pallas import tpu_sc as plsc

NC = 2
NS = 16
NW = NC * NS
F = 256
FH = F // 2
LANES = 16


def _make_segsum(n_nodes, n_edges):
  eps = n_edges // NS
  half = n_nodes // 2
  kc = 64
  cap = 10176
  sb = 2000
  nsb = eps // sb
  assert nsb * sb == eps
  npad = 5056
  wb = 64
  nq = npad // wb
  dump = half

  mesh = plsc.VectorSubcoreMesh(core_axis_name="c", subcore_axis_name="s",
                                num_cores=NC, num_subcores=NS)

  @functools.partial(
      pl.kernel,
      out_type=jax.ShapeDtypeStruct((NC, npad, F), jnp.float32),
      mesh=mesh,
      scratch_types=[
          pltpu.VMEM((cap,), jnp.int32),
          pltpu.VMEM((sb,), jnp.int32),
          pltpu.VMEM((sb,), jnp.int32),
          pltpu.VMEM((2, kc), jnp.int32),
          pltpu.VMEM((2, kc), jnp.int32),
          pltpu.VMEM((2, kc, F), jnp.float32),
          pltpu.VMEM_SHARED((npad, F), jnp.float32),
          pltpu.SemaphoreType.DMA,
          pltpu.SemaphoreType.DMA,
          pltpu.SemaphoreType.DMA,
          pltpu.SemaphoreType.DMA,
      ],
      compiler_params=pltpu.CompilerParams(use_tc_tiling_on_sc=False,
                                           needs_layout_passes=False),
  )
  def segsum(x, src, dst, out, pk, sv, dv, gb, db, rows, acc,
             semg0, semg1, sems0, sems1):
    c = lax.axis_index("c")
    s = lax.axis_index("s")
    semg = (semg0, semg1)
    sems = (sems0, sems1)
    lo = c * half

    dump16 = jnp.full((LANES,), dump, jnp.int32)

    @pl.loop(0, cap // LANES)
    def _fill(i):
      pk[pl.ds(i * LANES, LANES)] = dump16

    off = jnp.zeros((LANES,), jnp.int32)
    for blk in range(nsb):
      base = s * eps + blk * sb
      pltpu.sync_copy(src.at[pl.ds(base, sb)], sv)
      pltpu.sync_copy(dst.at[pl.ds(base, sb)], dv)

      def scan_iter(i, o):
        sv16 = sv[pl.ds(i * LANES, LANES)]
        ldst = dv[pl.ds(i * LANES, LANES)] - lo
        m = (ldst >= 0) & (ldst < half)
        packed = sv16 * 16384 + ldst
        plsc.store_compressed(pk.at[pl.ds(o[0], LANES)], packed, mask=m)
        return o + plsc.all_reduce_population_count(m)

      off = pl.loop(0, sb // LANES, init_carry=off)(scan_iter)
    nmatch = off[0]
    nt = lax.shift_right_logical(nmatch + (2 * kc - 1), 7)

    zero16 = jnp.zeros((LANES,), jnp.float32)
    for i in range(wb):
      for j in range(F // LANES):
        rows[0, i, pl.ds(j * LANES, LANES)] = zero16
    nq_per = -(-nq // NS)
    for j in range(nq_per):
      q = s + j * NS

      @pl.when(q < nq)
      def _z():
        pltpu.sync_copy(rows.at[0], acc.at[pl.ds(q * wb, wb)])

    plsc.subcore_barrier()

    def fire(jc, b):
      @pl.when(jc >= 2)
      def _drain():
        pltpu.make_async_copy(rows.at[b], acc.at[db.at[b]], sems[b]).wait()

      for i in range(kc // LANES):
        v = pk[pl.ds(jc * kc + i * LANES, LANES)]
        gb[b, pl.ds(i * LANES, LANES)] = lax.shift_right_logical(v, 14)
        db[b, pl.ds(i * LANES, LANES)] = v & 16383
      pltpu.async_copy(x.at[gb.at[b]], rows.at[b], semg[b])

    def consume(jc, b):
      pltpu.make_async_copy(x.at[gb.at[b]], rows.at[b], semg[b]).wait()
      pltpu.async_copy(rows.at[b], acc.at[db.at[b]], sems[b], add=True)

    fire(0, 0)

    @pl.loop(0, nt)
    def _chunk(t):
      fire(2 * t + 1, 1)
      consume(2 * t, 0)
      fire(2 * t + 2, 0)
      consume(2 * t + 1, 1)

    consume(2 * nt, 0)
    pltpu.make_async_copy(rows.at[0], acc.at[db.at[0]], sems[0]).wait()

    @pl.when(nt > 0)
    def _drain1():
      pltpu.make_async_copy(rows.at[1], acc.at[db.at[1]], sems[1]).wait()

    plsc.subcore_barrier()
    for j in range(nq_per):
      q = s + j * NS

      @pl.when(q < nq)
      def _w():
        pltpu.sync_copy(acc.at[pl.ds(q * wb, wb)], rows.at[0])
        pltpu.sync_copy(rows.at[0], out.at[c, pl.ds(q * wb, wb)])

  return segsum


def _make_decoder(n_edges):
  epw = n_edges // NW
  k = 40
  nch = epw // k
  assert nch * k == epw and nch % 2 == 1
  kpad = 48
  pw = 17

  mesh = plsc.VectorSubcoreMesh(core_axis_name="c", subcore_axis_name="s",
                                num_cores=NC, num_subcores=NS)

  @functools.partial(
      pl.kernel,
      out_type=jax.ShapeDtypeStruct((n_edges,), jnp.float32),
      mesh=mesh,
      scratch_types=[
          pltpu.VMEM((epw,), jnp.int32),
          pltpu.VMEM((epw,), jnp.int32),
          pltpu.VMEM((2, kpad, F), jnp.bfloat16),
          pltpu.VMEM((2, kpad, F), jnp.bfloat16),
          pltpu.VMEM((kpad, pw), jnp.float32),
          pltpu.VMEM((2, kpad), jnp.float32),
          pltpu.VMEM((F,), jnp.float32),
          pltpu.VMEM((F,), jnp.float32),
          pltpu.VMEM((LANES,), jnp.float32),
          pltpu.SemaphoreType.DMA,
          pltpu.SemaphoreType.DMA,
          pltpu.SemaphoreType.DMA,
          pltpu.SemaphoreType.DMA,
          pltpu.SemaphoreType.DMA,
          pltpu.SemaphoreType.DMA,
      ],
      compiler_params=pltpu.CompilerParams(use_tc_tiling_on_sc=False,
                                           needs_layout_passes=False),
  )
  def decoder(hu, hm, row, col, bd1, wd2, bd2, out,
              irv, icv, ru, rm, pacc, pv, b1v, w2v, b2v,
              semu0, semu1, semm0, semm1, semo0, semo1):
    c = lax.axis_index("c")
    s = lax.axis_index("s")
    w = s * NC + c
    semu = (semu0, semu1)
    semm = (semm0, semm1)
    semo = (semo0, semo1)

    base0 = w * epw
    pltpu.sync_copy(row.at[pl.ds(base0, epw)], irv)
    pltpu.sync_copy(col.at[pl.ds(base0, epw)], icv)
    pltpu.sync_copy(bd1, b1v)
    pltpu.sync_copy(wd2, w2v)
    pltpu.sync_copy(bd2, b2v)
    b1 = [b1v[pl.ds(LANES * j, LANES)] for j in range(F // LANES)]
    w2 = [w2v[pl.ds(LANES * j, LANES)] for j in range(F // LANES)]
    b2 = b2v[pl.ds(0, LANES)]
    lane = lax.iota(jnp.int32, LANES)

    def fire(jc, b):
      pltpu.async_copy(hu.at[irv.at[pl.ds(jc * k, k)]],
                       ru.at[b, pl.ds(0, k)], semu[b])
      pltpu.async_copy(hm.at[icv.at[pl.ds(jc * k, k)]],
                       rm.at[b, pl.ds(0, k)], semm[b])

    def consume(jc, b):
      pltpu.make_async_copy(hu.at[irv.at[pl.ds(jc * k, k)]],
                            ru.at[b, pl.ds(0, k)], semu[b]).wait()
      pltpu.make_async_copy(hm.at[icv.at[pl.ds(jc * k, k)]],
                            rm.at[b, pl.ds(0, k)], semm[b]).wait()

      @pl.loop(0, kpad, unroll=2)
      def _edge(e):
        a = [jnp.zeros((LANES,), jnp.float32) for _ in range(4)]
        for j8 in range(F // (2 * LANES)):
          tu = ru[b, e, pl.ds(2 * LANES * j8, 2 * LANES)]
          tm = rm[b, e, pl.ds(2 * LANES * j8, 2 * LANES)]
          ue, uo = plsc.unpack(tu, format=plsc.PackFormat.INTERLEAVED)
          me, mo = plsc.unpack(tm, format=plsc.PackFormat.INTERLEAVED)
          for half, (uu, mm) in enumerate(((ue, me), (uo, mo))):
            j = 2 * j8 + half
            t = jnp.maximum(uu + mm + b1[j], 0.0)
            a[j % 4] = a[j % 4] + t * w2[j]
        pacc[e, pl.ds(0, LANES)] = (a[0] + a[1]) + (a[2] + a[3])

      for g in range(kpad // LANES):
        rowv = lane + g * LANES
        r = [jnp.zeros((LANES,), jnp.float32) for _ in range(4)]
        for l in range(LANES):
          col_l = jnp.broadcast_to(jnp.int32(l), (LANES,))
          r[l % 4] = r[l % 4] + plsc.load_gather(pacc, [rowv, col_l])
        tot = (r[0] + r[1]) + (r[2] + r[3])
        pv[b, pl.ds(g * LANES, LANES)] = 5.0 / (1.0 + jnp.exp(-(tot + b2)))

      @pl.when(jc >= 2)
      def _drain():
        pltpu.make_async_copy(pv.at[b, pl.ds(0, k)],
                              out.at[pl.ds(base0 + (jc - 2) * k, k)],
                              semo[b]).wait()

      pltpu.async_copy(pv.at[b, pl.ds(0, k)],
                       out.at[pl.ds(base0 + jc * k, k)], semo[b])

    fire(0, 0)

    @pl.loop(0, nch // 2)
    def _chunk(t):
      fire(2 * t + 1, 1)
      consume(2 * t, 0)
      fire(2 * t + 2, 0)
      consume(2 * t + 1, 1)

    consume(nch - 1, 0)
    pltpu.make_async_copy(pv.at[0, pl.ds(0, k)],
                          out.at[pl.ds(base0 + (nch - 1) * k, k)],
                          semo[0]).wait()
    pltpu.make_async_copy(pv.at[1, pl.ds(0, k)],
                          out.at[pl.ds(base0 + (nch - 2) * k, k)],
                          semo[1]).wait()

  return decoder


def _dense_body(agg, x, wl, wr, b, g, be, z_ref):
  out = jnp.dot(agg[...], wl[...], preferred_element_type=jnp.float32)
  out = out + jnp.dot(x[...], wr[...], preferred_element_type=jnp.float32)
  out = out + b[...]
  n = jnp.sqrt(jnp.sum(out * out, axis=1, keepdims=True))
  rn = out / jnp.maximum(n, 1e-12)
  m = jnp.mean(rn, axis=0, keepdims=True)
  v = jnp.mean((rn - m) ** 2, axis=0, keepdims=True)
  z = (rn - m) / jnp.sqrt(v + 1e-5) * g[...] + be[...]
  z_ref[...] = jnp.maximum(z, 0.0)


def _dense_h_body(agg, x, wl, wr, b, g, be, wh, h_ref):
  out = jnp.dot(agg[...], wl[...], preferred_element_type=jnp.float32)
  out = out + jnp.dot(x[...], wr[...], preferred_element_type=jnp.float32)
  out = out + b[...]
  n = jnp.sqrt(jnp.sum(out * out, axis=1, keepdims=True))
  rn = out / jnp.maximum(n, 1e-12)
  m = jnp.mean(rn, axis=0, keepdims=True)
  v = jnp.mean((rn - m) ** 2, axis=0, keepdims=True)
  z = (rn - m) / jnp.sqrt(v + 1e-5) * g[...] + be[...]
  h = jnp.dot(z, wh[...], preferred_element_type=jnp.float32)
  h_ref[...] = h.astype(jnp.bfloat16)


def _dense_relu(agg, x, wl, wr, bsum, g, be):
  n = x.shape[0]
  return pl.pallas_call(
      _dense_body,
      out_shape=jax.ShapeDtypeStruct((n, F), jnp.float32),
  )(agg, x, wl, wr, bsum.reshape(1, F), g.reshape(1, F), be.reshape(1, F))


def _dense_h(agg, x, wl, wr, bsum, g, be, wh):
  n = x.shape[0]
  return pl.pallas_call(
      _dense_h_body,
      out_shape=jax.ShapeDtypeStruct((n, F), jnp.bfloat16),
  )(agg, x, wl, wr, bsum.reshape(1, F), g.reshape(1, F), be.reshape(1, F), wh)


def kernel(x_user, x_movie, ei_um, ei_mu, edge_label_index,
           Wl1_um, bl1_um, Wr1_um, br1_um,
           Wl2_um, bl2_um, Wr2_um, br2_um,
           Wl1_mu, bl1_mu, Wr1_mu, br1_mu,
           Wl2_mu, bl2_mu, Wr2_mu, br2_mu,
           g1_user, be1_user, g2_user, be2_user,
           g1_movie, be1_movie, g2_movie, be2_movie,
           Wd1, bd1, Wd2, bd2):
  nu = x_user.shape[0]
  nm = x_movie.shape[0]
  e = ei_um.shape[1]
  el = edge_label_index.shape[1]

  segsum = _make_segsum(nu, e)
  decoder = _make_decoder(el)
  half = nu // 2

  def agg_of(o):
    return o[:, :half, :].reshape(nu, F)

  a1m = agg_of(segsum(x_user, ei_um[0], ei_um[1]))
  a1u = agg_of(segsum(x_movie, ei_mu[0], ei_mu[1]))
  z_movie = _dense_relu(a1m, x_movie, Wl1_um, Wr1_um,
                        bl1_um + br1_um, g1_movie, be1_movie)
  z_user = _dense_relu(a1u, x_user, Wl1_mu, Wr1_mu,
                       bl1_mu + br1_mu, g1_user, be1_user)

  a2m = agg_of(segsum(z_user, ei_um[0], ei_um[1]))
  a2u = agg_of(segsum(z_movie, ei_mu[0], ei_mu[1]))
  hu = _dense_h(a2u, z_user, Wl2_mu, Wr2_mu,
                bl2_mu + br2_mu, g2_user, be2_user, Wd1[:F])
  hm = _dense_h(a2m, z_movie, Wl2_um, Wr2_um,
                bl2_um + br2_um, g2_movie, be2_movie, Wd1[F:])

  blk = jnp.arange(F).reshape(-1, 2 * LANES)
  perm = jnp.concatenate([blk[:, 0::2], blk[:, 1::2]], axis=1).reshape(-1)
  bd2p = jnp.full((LANES,), bd2[0], jnp.float32)
  return decoder(hu, hm, edge_label_index[0], edge_label_index[1],
                 bd1[perm], Wd2.reshape(F)[perm], bd2p)


# speedup vs baseline: 1.8666x; 1.8666x over previous
import functools

import jax
import jax.numpy as jnp
from jax import lax
from jax.experimental import pallas as pl
from jax.experimental.pallas import tpu as pltpu
from jax.experimental.pallas import tpu_sc as plsc

NC = 2
NS = 16
NW = NC * NS
F = 256
FH = F // 2
LANES = 16


def _make_segsum(n_nodes, n_edges):
  eps = n_edges // NS
  k = 80
  nch = eps // k
  assert nch * k == eps and nch % 2 == 1
  wb = 80
  nq = n_nodes // wb
  assert nq * wb == n_nodes

  mesh = plsc.VectorSubcoreMesh(core_axis_name="c", subcore_axis_name="s",
                                num_cores=NC, num_subcores=NS)

  @functools.partial(
      pl.kernel,
      out_type=jax.ShapeDtypeStruct((NC, n_nodes, FH), jnp.float32),
      mesh=mesh,
      scratch_types=[
          pltpu.VMEM((eps,), jnp.int32),
          pltpu.VMEM((2, k), jnp.int32),
          pltpu.VMEM((2, k, FH), jnp.float32),
          pltpu.VMEM_SHARED((n_nodes, FH), jnp.float32),
          pltpu.SemaphoreType.DMA,
          pltpu.SemaphoreType.DMA,
          pltpu.SemaphoreType.DMA,
          pltpu.SemaphoreType.DMA,
          pltpu.SemaphoreType.DMA,
          pltpu.SemaphoreType.DMA,
      ],
  )
  def segsum(x2, src, dst, out, giv, idb, rows, acc, semg0, semg1, semi0,
             semi1, sems0, sems1):
    c = lax.axis_index("c")
    s = lax.axis_index("s")
    semg = (semg0, semg1)
    semi = (semi0, semi1)
    sems = (sems0, sems1)

    pltpu.sync_copy(src.at[pl.ds(s * eps, eps)], giv)

    @pl.loop(0, eps // LANES)
    def _xform(i):
      giv[pl.ds(i * LANES, LANES)] = giv[pl.ds(i * LANES, LANES)] * 2 + c

    zero16 = jnp.zeros((LANES,), jnp.float32)
    for i in range(wb):
      for j in range(FH // LANES):
        rows[0, i, pl.ds(j * LANES, LANES)] = zero16
    nq_per = -(-nq // NS)
    for j in range(nq_per):
      q = s + j * NS

      @pl.when(q < nq)
      def _z():
        pltpu.sync_copy(rows.at[0], acc.at[pl.ds(q * wb, wb)])

    plsc.subcore_barrier()

    def fire(jc, b):
      @pl.when(jc >= 2)
      def _drain():
        pltpu.make_async_copy(rows.at[b], acc.at[idb.at[b]], sems[b]).wait()

      pltpu.async_copy(dst.at[pl.ds(s * eps + jc * k, k)], idb.at[b], semi[b])
      pltpu.async_copy(x2.at[giv.at[pl.ds(jc * k, k)]], rows.at[b], semg[b])

    def consume(jc, b):
      pltpu.make_async_copy(dst.at[pl.ds(s * eps + jc * k, k)], idb.at[b],
                            semi[b]).wait()
      pltpu.make_async_copy(x2.at[giv.at[pl.ds(jc * k, k)]], rows.at[b],
                            semg[b]).wait()
      pltpu.async_copy(rows.at[b], acc.at[idb.at[b]], sems[b], add=True)

    fire(0, 0)

    @pl.loop(0, nch // 2)
    def _chunk(t):
      fire(2 * t + 1, 1)
      consume(2 * t, 0)
      fire(2 * t + 2, 0)
      consume(2 * t + 1, 1)

    consume(nch - 1, 0)
    pltpu.make_async_copy(rows.at[0], acc.at[idb.at[0]], sems[0]).wait()
    pltpu.make_async_copy(rows.at[1], acc.at[idb.at[1]], sems[1]).wait()

    plsc.subcore_barrier()
    for j in range(nq_per):
      q = s + j * NS

      @pl.when(q < nq)
      def _w():
        pltpu.sync_copy(acc.at[pl.ds(q * wb, wb)], rows.at[0])
        pltpu.sync_copy(rows.at[0], out.at[c, pl.ds(q * wb, wb)])

  return segsum


def _make_decoder(n_edges):
  epw = n_edges // NW
  k = 40
  nch = epw // k
  assert nch * k == epw and nch % 2 == 1
  kpad = 48
  pw = 17

  mesh = plsc.VectorSubcoreMesh(core_axis_name="c", subcore_axis_name="s",
                                num_cores=NC, num_subcores=NS)

  @functools.partial(
      pl.kernel,
      out_type=jax.ShapeDtypeStruct((n_edges,), jnp.float32),
      mesh=mesh,
      scratch_types=[
          pltpu.VMEM((epw,), jnp.int32),
          pltpu.VMEM((epw,), jnp.int32),
          pltpu.VMEM((2, kpad, F), jnp.bfloat16),
          pltpu.VMEM((2, kpad, F), jnp.bfloat16),
          pltpu.VMEM((kpad, pw), jnp.float32),
          pltpu.VMEM((2, kpad), jnp.float32),
          pltpu.VMEM((F,), jnp.float32),
          pltpu.VMEM((F,), jnp.float32),
          pltpu.VMEM((LANES,), jnp.float32),
          pltpu.SemaphoreType.DMA,
          pltpu.SemaphoreType.DMA,
          pltpu.SemaphoreType.DMA,
          pltpu.SemaphoreType.DMA,
          pltpu.SemaphoreType.DMA,
          pltpu.SemaphoreType.DMA,
      ],
      compiler_params=pltpu.CompilerParams(use_tc_tiling_on_sc=False,
                                           needs_layout_passes=False),
  )
  def decoder(hu, hm, row, col, bd1, wd2, bd2, out,
              irv, icv, ru, rm, pacc, pv, b1v, w2v, b2v,
              semu0, semu1, semm0, semm1, semo0, semo1):
    c = lax.axis_index("c")
    s = lax.axis_index("s")
    w = s * NC + c
    semu = (semu0, semu1)
    semm = (semm0, semm1)
    semo = (semo0, semo1)

    base0 = w * epw
    pltpu.sync_copy(row.at[pl.ds(base0, epw)], irv)
    pltpu.sync_copy(col.at[pl.ds(base0, epw)], icv)
    pltpu.sync_copy(bd1, b1v)
    pltpu.sync_copy(wd2, w2v)
    pltpu.sync_copy(bd2, b2v)
    b1 = [b1v[pl.ds(LANES * j, LANES)] for j in range(F // LANES)]
    w2 = [w2v[pl.ds(LANES * j, LANES)] for j in range(F // LANES)]
    b2 = b2v[pl.ds(0, LANES)]
    lane = lax.iota(jnp.int32, LANES)

    def fire(jc, b):
      pltpu.async_copy(hu.at[irv.at[pl.ds(jc * k, k)]],
                       ru.at[b, pl.ds(0, k)], semu[b])
      pltpu.async_copy(hm.at[icv.at[pl.ds(jc * k, k)]],
                       rm.at[b, pl.ds(0, k)], semm[b])

    def consume(jc, b):
      pltpu.make_async_copy(hu.at[irv.at[pl.ds(jc * k, k)]],
                            ru.at[b, pl.ds(0, k)], semu[b]).wait()
      pltpu.make_async_copy(hm.at[icv.at[pl.ds(jc * k, k)]],
                            rm.at[b, pl.ds(0, k)], semm[b]).wait()

      @pl.loop(0, kpad, unroll=2)
      def _edge(e):
        a = [jnp.zeros((LANES,), jnp.float32) for _ in range(4)]
        for j8 in range(F // (2 * LANES)):
          tu = ru[b, e, pl.ds(2 * LANES * j8, 2 * LANES)]
          tm = rm[b, e, pl.ds(2 * LANES * j8, 2 * LANES)]
          ue, uo = plsc.unpack(tu, format=plsc.PackFormat.INTERLEAVED)
          me, mo = plsc.unpack(tm, format=plsc.PackFormat.INTERLEAVED)
          for half, (uu, mm) in enumerate(((ue, me), (uo, mo))):
            j = 2 * j8 + half
            t = jnp.maximum(uu + mm + b1[j], 0.0)
            a[j % 4] = a[j % 4] + t * w2[j]
        pacc[e, pl.ds(0, LANES)] = (a[0] + a[1]) + (a[2] + a[3])

      for g in range(kpad // LANES):
        rowv = lane + g * LANES
        r = [jnp.zeros((LANES,), jnp.float32) for _ in range(4)]
        for l in range(LANES):
          col_l = jnp.broadcast_to(jnp.int32(l), (LANES,))
          r[l % 4] = r[l % 4] + plsc.load_gather(pacc, [rowv, col_l])
        tot = (r[0] + r[1]) + (r[2] + r[3])
        pv[b, pl.ds(g * LANES, LANES)] = 5.0 / (1.0 + jnp.exp(-(tot + b2)))

      @pl.when(jc >= 2)
      def _drain():
        pltpu.make_async_copy(pv.at[b, pl.ds(0, k)],
                              out.at[pl.ds(base0 + (jc - 2) * k, k)],
                              semo[b]).wait()

      pltpu.async_copy(pv.at[b, pl.ds(0, k)],
                       out.at[pl.ds(base0 + jc * k, k)], semo[b])

    fire(0, 0)

    @pl.loop(0, nch // 2)
    def _chunk(t):
      fire(2 * t + 1, 1)
      consume(2 * t, 0)
      fire(2 * t + 2, 0)
      consume(2 * t + 1, 1)

    consume(nch - 1, 0)
    pltpu.make_async_copy(pv.at[0, pl.ds(0, k)],
                          out.at[pl.ds(base0 + (nch - 1) * k, k)],
                          semo[0]).wait()
    pltpu.make_async_copy(pv.at[1, pl.ds(0, k)],
                          out.at[pl.ds(base0 + (nch - 2) * k, k)],
                          semo[1]).wait()

  return decoder


def _dense_body(a0, a1, x, wl0, wl1, wr, b, g, be, z_ref):
  out = jnp.dot(a0[...], wl0[...], preferred_element_type=jnp.float32)
  out = out + jnp.dot(a1[...], wl1[...], preferred_element_type=jnp.float32)
  out = out + jnp.dot(x[...], wr[...], preferred_element_type=jnp.float32)
  out = out + b[...]
  n = jnp.sqrt(jnp.sum(out * out, axis=1, keepdims=True))
  rn = out / jnp.maximum(n, 1e-12)
  m = jnp.mean(rn, axis=0, keepdims=True)
  v = jnp.mean((rn - m) ** 2, axis=0, keepdims=True)
  z = (rn - m) / jnp.sqrt(v + 1e-5) * g[...] + be[...]
  z_ref[...] = jnp.maximum(z, 0.0)


def _dense_h_body(a0, a1, x, wl0, wl1, wr, b, g, be, wh, h_ref):
  out = jnp.dot(a0[...], wl0[...], preferred_element_type=jnp.float32)
  out = out + jnp.dot(a1[...], wl1[...], preferred_element_type=jnp.float32)
  out = out + jnp.dot(x[...], wr[...], preferred_element_type=jnp.float32)
  out = out + b[...]
  n = jnp.sqrt(jnp.sum(out * out, axis=1, keepdims=True))
  rn = out / jnp.maximum(n, 1e-12)
  m = jnp.mean(rn, axis=0, keepdims=True)
  v = jnp.mean((rn - m) ** 2, axis=0, keepdims=True)
  z = (rn - m) / jnp.sqrt(v + 1e-5) * g[...] + be[...]
  h = jnp.dot(z, wh[...], preferred_element_type=jnp.float32)
  h_ref[...] = h.astype(jnp.bfloat16)


def _dense_relu(a0, a1, x, wl, wr, bsum, g, be):
  n = x.shape[0]
  return pl.pallas_call(
      _dense_body,
      out_shape=jax.ShapeDtypeStruct((n, F), jnp.float32),
  )(a0, a1, x, wl[:FH], wl[FH:], wr, bsum.reshape(1, F),
    g.reshape(1, F), be.reshape(1, F))


def _dense_h(a0, a1, x, wl, wr, bsum, g, be, wh):
  n = x.shape[0]
  return pl.pallas_call(
      _dense_h_body,
      out_shape=jax.ShapeDtypeStruct((n, F), jnp.bfloat16),
  )(a0, a1, x, wl[:FH], wl[FH:], wr, bsum.reshape(1, F),
    g.reshape(1, F), be.reshape(1, F), wh)


def kernel(x_user, x_movie, ei_um, ei_mu, edge_label_index,
           Wl1_um, bl1_um, Wr1_um, br1_um,
           Wl2_um, bl2_um, Wr2_um, br2_um,
           Wl1_mu, bl1_mu, Wr1_mu, br1_mu,
           Wl2_mu, bl2_mu, Wr2_mu, br2_mu,
           g1_user, be1_user, g2_user, be2_user,
           g1_movie, be1_movie, g2_movie, be2_movie,
           Wd1, bd1, Wd2, bd2):
  nu = x_user.shape[0]
  nm = x_movie.shape[0]
  e = ei_um.shape[1]
  el = edge_label_index.shape[1]

  segsum = _make_segsum(nu, e)
  decoder = _make_decoder(el)

  a1m = segsum(x_user.reshape(nu * 2, FH), ei_um[0], ei_um[1])
  a1u = segsum(x_movie.reshape(nm * 2, FH), ei_mu[0], ei_mu[1])
  z_movie = _dense_relu(a1m[0], a1m[1], x_movie, Wl1_um, Wr1_um,
                        bl1_um + br1_um, g1_movie, be1_movie)
  z_user = _dense_relu(a1u[0], a1u[1], x_user, Wl1_mu, Wr1_mu,
                       bl1_mu + br1_mu, g1_user, be1_user)

  a2m = segsum(z_user.reshape(nu * 2, FH), ei_um[0], ei_um[1])
  a2u = segsum(z_movie.reshape(nm * 2, FH), ei_mu[0], ei_mu[1])
  hu = _dense_h(a2u[0], a2u[1], z_user, Wl2_mu, Wr2_mu,
                bl2_mu + br2_mu, g2_user, be2_user, Wd1[:F])
  hm = _dense_h(a2m[0], a2m[1], z_movie, Wl2_um, Wr2_um,
                bl2_um + br2_um, g2_movie, be2_movie, Wd1[F:])

  blk = jnp.arange(F).reshape(-1, 2 * LANES)
  perm = jnp.concatenate([blk[:, 0::2], blk[:, 1::2]], axis=1).reshape(-1)
  bd2p = jnp.full((LANES,), bd2[0], jnp.float32)
  return decoder(hu, hm, edge_label_index[0], edge_label_index[1],
                 bd1[perm], Wd2.reshape(F)[perm], bd2p)


# speedup vs baseline: 2.1959x; 1.1764x over previous
import functools

import jax
import jax.numpy as jnp
from jax import lax
from jax.experimental import pallas as pl
from jax.experimental.pallas import tpu as pltpu
from jax.experimental.pallas import tpu_sc as plsc

NC = 2
NS = 16
NW = NC * NS
F = 256
FH = F // 2
LANES = 16


def _make_segsum(n_nodes, n_edges):
  eps = n_edges // NS
  k = 80
  nch = eps // k
  assert nch * k == eps and nch % 2 == 1
  wb = 80
  nq = n_nodes // wb
  assert nq * wb == n_nodes

  mesh = plsc.VectorSubcoreMesh(core_axis_name="c", subcore_axis_name="s",
                                num_cores=NC, num_subcores=NS)

  @functools.partial(
      pl.kernel,
      out_type=jax.ShapeDtypeStruct((NC, n_nodes, FH), jnp.bfloat16),
      mesh=mesh,
      scratch_types=[
          pltpu.VMEM((eps,), jnp.int32),
          pltpu.VMEM((2, k), jnp.int32),
          pltpu.VMEM((2, k, FH), jnp.bfloat16),
          pltpu.VMEM_SHARED((n_nodes, FH), jnp.bfloat16),
          pltpu.SemaphoreType.DMA,
          pltpu.SemaphoreType.DMA,
          pltpu.SemaphoreType.DMA,
          pltpu.SemaphoreType.DMA,
          pltpu.SemaphoreType.DMA,
          pltpu.SemaphoreType.DMA,
      ],
      compiler_params=pltpu.CompilerParams(use_tc_tiling_on_sc=False,
                                           needs_layout_passes=False),
  )
  def segsum(x2, src, dst, out, giv, idb, rows, acc, semg0, semg1, semi0,
             semi1, sems0, sems1):
    c = lax.axis_index("c")
    s = lax.axis_index("s")
    semg = (semg0, semg1)
    semi = (semi0, semi1)
    sems = (sems0, sems1)

    pltpu.sync_copy(src.at[pl.ds(s * eps, eps)], giv)

    @pl.loop(0, eps // LANES)
    def _xform(i):
      giv[pl.ds(i * LANES, LANES)] = giv[pl.ds(i * LANES, LANES)] * 2 + c

    zero32 = jnp.zeros((2 * LANES,), jnp.bfloat16)
    for i in range(wb):
      for j in range(FH // (2 * LANES)):
        rows[0, i, pl.ds(j * 2 * LANES, 2 * LANES)] = zero32
    nq_per = -(-nq // NS)
    for j in range(nq_per):
      q = s + j * NS

      @pl.when(q < nq)
      def _z():
        pltpu.sync_copy(rows.at[0], acc.at[pl.ds(q * wb, wb)])

    plsc.subcore_barrier()

    def fire(jc, b):
      @pl.when(jc >= 2)
      def _drain():
        pltpu.make_async_copy(rows.at[b], acc.at[idb.at[b]], sems[b]).wait()

      pltpu.async_copy(dst.at[pl.ds(s * eps + jc * k, k)], idb.at[b], semi[b])
      pltpu.async_copy(x2.at[giv.at[pl.ds(jc * k, k)]], rows.at[b], semg[b])

    def consume(jc, b):
      pltpu.make_async_copy(dst.at[pl.ds(s * eps + jc * k, k)], idb.at[b],
                            semi[b]).wait()
      pltpu.make_async_copy(x2.at[giv.at[pl.ds(jc * k, k)]], rows.at[b],
                            semg[b]).wait()
      pltpu.async_copy(rows.at[b], acc.at[idb.at[b]], sems[b], add=True)

    fire(0, 0)

    @pl.loop(0, nch // 2)
    def _chunk(t):
      fire(2 * t + 1, 1)
      consume(2 * t, 0)
      fire(2 * t + 2, 0)
      consume(2 * t + 1, 1)

    consume(nch - 1, 0)
    pltpu.make_async_copy(rows.at[0], acc.at[idb.at[0]], sems[0]).wait()
    pltpu.make_async_copy(rows.at[1], acc.at[idb.at[1]], sems[1]).wait()

    plsc.subcore_barrier()
    for j in range(nq_per):
      q = s + j * NS

      @pl.when(q < nq)
      def _w():
        pltpu.sync_copy(acc.at[pl.ds(q * wb, wb)], rows.at[0])
        pltpu.sync_copy(rows.at[0], out.at[c, pl.ds(q * wb, wb)])

  return segsum


def _make_decoder(n_edges):
  epw = n_edges // NW
  k = 40
  nch = epw // k
  assert nch * k == epw and nch % 2 == 1
  kpad = 48
  pw = 17

  mesh = plsc.VectorSubcoreMesh(core_axis_name="c", subcore_axis_name="s",
                                num_cores=NC, num_subcores=NS)

  @functools.partial(
      pl.kernel,
      out_type=jax.ShapeDtypeStruct((n_edges,), jnp.float32),
      mesh=mesh,
      scratch_types=[
          pltpu.VMEM((epw,), jnp.int32),
          pltpu.VMEM((epw,), jnp.int32),
          pltpu.VMEM((2, kpad, F), jnp.bfloat16),
          pltpu.VMEM((2, kpad, F), jnp.bfloat16),
          pltpu.VMEM((kpad, pw), jnp.float32),
          pltpu.VMEM((2, kpad), jnp.float32),
          pltpu.VMEM((F,), jnp.float32),
          pltpu.VMEM((F,), jnp.float32),
          pltpu.VMEM((LANES,), jnp.float32),
          pltpu.SemaphoreType.DMA,
          pltpu.SemaphoreType.DMA,
          pltpu.SemaphoreType.DMA,
          pltpu.SemaphoreType.DMA,
          pltpu.SemaphoreType.DMA,
          pltpu.SemaphoreType.DMA,
      ],
      compiler_params=pltpu.CompilerParams(use_tc_tiling_on_sc=False,
                                           needs_layout_passes=False),
  )
  def decoder(hu, hm, row, col, bd1, wd2, bd2, out,
              irv, icv, ru, rm, pacc, pv, b1v, w2v, b2v,
              semu0, semu1, semm0, semm1, semo0, semo1):
    c = lax.axis_index("c")
    s = lax.axis_index("s")
    w = s * NC + c
    semu = (semu0, semu1)
    semm = (semm0, semm1)
    semo = (semo0, semo1)

    base0 = w * epw
    pltpu.sync_copy(row.at[pl.ds(base0, epw)], irv)
    pltpu.sync_copy(col.at[pl.ds(base0, epw)], icv)
    pltpu.sync_copy(bd1, b1v)
    pltpu.sync_copy(wd2, w2v)
    pltpu.sync_copy(bd2, b2v)
    b1 = [b1v[pl.ds(LANES * j, LANES)] for j in range(F // LANES)]
    w2 = [w2v[pl.ds(LANES * j, LANES)] for j in range(F // LANES)]
    b2 = b2v[pl.ds(0, LANES)]
    lane = lax.iota(jnp.int32, LANES)

    def fire(jc, b):
      pltpu.async_copy(hu.at[irv.at[pl.ds(jc * k, k)]],
                       ru.at[b, pl.ds(0, k)], semu[b])
      pltpu.async_copy(hm.at[icv.at[pl.ds(jc * k, k)]],
                       rm.at[b, pl.ds(0, k)], semm[b])

    def consume(jc, b):
      pltpu.make_async_copy(hu.at[irv.at[pl.ds(jc * k, k)]],
                            ru.at[b, pl.ds(0, k)], semu[b]).wait()
      pltpu.make_async_copy(hm.at[icv.at[pl.ds(jc * k, k)]],
                            rm.at[b, pl.ds(0, k)], semm[b]).wait()

      @pl.loop(0, kpad, unroll=2)
      def _edge(e):
        a = [jnp.zeros((LANES,), jnp.float32) for _ in range(4)]
        for j8 in range(F // (2 * LANES)):
          tu = ru[b, e, pl.ds(2 * LANES * j8, 2 * LANES)]
          tm = rm[b, e, pl.ds(2 * LANES * j8, 2 * LANES)]
          ue, uo = plsc.unpack(tu, format=plsc.PackFormat.INTERLEAVED)
          me, mo = plsc.unpack(tm, format=plsc.PackFormat.INTERLEAVED)
          for half, (uu, mm) in enumerate(((ue, me), (uo, mo))):
            j = 2 * j8 + half
            t = jnp.maximum(uu + mm + b1[j], 0.0)
            a[j % 4] = a[j % 4] + t * w2[j]
        pacc[e, pl.ds(0, LANES)] = (a[0] + a[1]) + (a[2] + a[3])

      for g in range(kpad // LANES):
        rowv = lane + g * LANES
        r = [jnp.zeros((LANES,), jnp.float32) for _ in range(4)]
        for l in range(LANES):
          col_l = jnp.broadcast_to(jnp.int32(l), (LANES,))
          r[l % 4] = r[l % 4] + plsc.load_gather(pacc, [rowv, col_l])
        tot = (r[0] + r[1]) + (r[2] + r[3])
        pv[b, pl.ds(g * LANES, LANES)] = 5.0 / (1.0 + jnp.exp(-(tot + b2)))

      @pl.when(jc >= 2)
      def _drain():
        pltpu.make_async_copy(pv.at[b, pl.ds(0, k)],
                              out.at[pl.ds(base0 + (jc - 2) * k, k)],
                              semo[b]).wait()

      pltpu.async_copy(pv.at[b, pl.ds(0, k)],
                       out.at[pl.ds(base0 + jc * k, k)], semo[b])

    fire(0, 0)

    @pl.loop(0, nch // 2)
    def _chunk(t):
      fire(2 * t + 1, 1)
      consume(2 * t, 0)
      fire(2 * t + 2, 0)
      consume(2 * t + 1, 1)

    consume(nch - 1, 0)
    pltpu.make_async_copy(pv.at[0, pl.ds(0, k)],
                          out.at[pl.ds(base0 + (nch - 1) * k, k)],
                          semo[0]).wait()
    pltpu.make_async_copy(pv.at[1, pl.ds(0, k)],
                          out.at[pl.ds(base0 + (nch - 2) * k, k)],
                          semo[1]).wait()

  return decoder


def _dense_body(a0, a1, x, wl0, wl1, wr, b, g, be, z_ref):
  out = jnp.dot(a0[...], wl0[...], preferred_element_type=jnp.float32)
  out = out + jnp.dot(a1[...], wl1[...], preferred_element_type=jnp.float32)
  out = out + jnp.dot(x[...], wr[...], preferred_element_type=jnp.float32)
  out = out + b[...]
  n = jnp.sqrt(jnp.sum(out * out, axis=1, keepdims=True))
  rn = out / jnp.maximum(n, 1e-12)
  m = jnp.mean(rn, axis=0, keepdims=True)
  v = jnp.mean((rn - m) ** 2, axis=0, keepdims=True)
  z = (rn - m) / jnp.sqrt(v + 1e-5) * g[...] + be[...]
  z_ref[...] = jnp.maximum(z, 0.0)


def _dense_h_body(a0, a1, x, wl0, wl1, wr, b, g, be, wh, h_ref):
  out = jnp.dot(a0[...], wl0[...], preferred_element_type=jnp.float32)
  out = out + jnp.dot(a1[...], wl1[...], preferred_element_type=jnp.float32)
  out = out + jnp.dot(x[...], wr[...], preferred_element_type=jnp.float32)
  out = out + b[...]
  n = jnp.sqrt(jnp.sum(out * out, axis=1, keepdims=True))
  rn = out / jnp.maximum(n, 1e-12)
  m = jnp.mean(rn, axis=0, keepdims=True)
  v = jnp.mean((rn - m) ** 2, axis=0, keepdims=True)
  z = (rn - m) / jnp.sqrt(v + 1e-5) * g[...] + be[...]
  h = jnp.dot(z, wh[...], preferred_element_type=jnp.float32)
  h_ref[...] = h.astype(jnp.bfloat16)


def _dense_relu(a0, a1, x, wl, wr, bsum, g, be):
  n = x.shape[0]
  return pl.pallas_call(
      _dense_body,
      out_shape=jax.ShapeDtypeStruct((n, F), jnp.float32),
  )(a0, a1, x, wl[:FH], wl[FH:], wr, bsum.reshape(1, F),
    g.reshape(1, F), be.reshape(1, F))


def _dense_h(a0, a1, x, wl, wr, bsum, g, be, wh):
  n = x.shape[0]
  return pl.pallas_call(
      _dense_h_body,
      out_shape=jax.ShapeDtypeStruct((n, F), jnp.bfloat16),
  )(a0, a1, x, wl[:FH], wl[FH:], wr, bsum.reshape(1, F),
    g.reshape(1, F), be.reshape(1, F), wh)


def kernel(x_user, x_movie, ei_um, ei_mu, edge_label_index,
           Wl1_um, bl1_um, Wr1_um, br1_um,
           Wl2_um, bl2_um, Wr2_um, br2_um,
           Wl1_mu, bl1_mu, Wr1_mu, br1_mu,
           Wl2_mu, bl2_mu, Wr2_mu, br2_mu,
           g1_user, be1_user, g2_user, be2_user,
           g1_movie, be1_movie, g2_movie, be2_movie,
           Wd1, bd1, Wd2, bd2):
  nu = x_user.shape[0]
  nm = x_movie.shape[0]
  e = ei_um.shape[1]
  el = edge_label_index.shape[1]

  segsum = _make_segsum(nu, e)
  decoder = _make_decoder(el)

  xub = x_user.astype(jnp.bfloat16)
  xmb = x_movie.astype(jnp.bfloat16)
  a1m = segsum(xub.reshape(nu * 2, FH), ei_um[0], ei_um[1])
  a1u = segsum(xmb.reshape(nm * 2, FH), ei_mu[0], ei_mu[1])
  z_movie = _dense_relu(a1m[0], a1m[1], x_movie, Wl1_um, Wr1_um,
                        bl1_um + br1_um, g1_movie, be1_movie)
  z_user = _dense_relu(a1u[0], a1u[1], x_user, Wl1_mu, Wr1_mu,
                       bl1_mu + br1_mu, g1_user, be1_user)

  a2m = segsum(z_user.astype(jnp.bfloat16).reshape(nu * 2, FH),
               ei_um[0], ei_um[1])
  a2u = segsum(z_movie.astype(jnp.bfloat16).reshape(nm * 2, FH),
               ei_mu[0], ei_mu[1])
  hu = _dense_h(a2u[0], a2u[1], z_user, Wl2_mu, Wr2_mu,
                bl2_mu + br2_mu, g2_user, be2_user, Wd1[:F])
  hm = _dense_h(a2m[0], a2m[1], z_movie, Wl2_um, Wr2_um,
                bl2_um + br2_um, g2_movie, be2_movie, Wd1[F:])

  blk = jnp.arange(F).reshape(-1, 2 * LANES)
  perm = jnp.concatenate([blk[:, 0::2], blk[:, 1::2]], axis=1).reshape(-1)
  bd2p = jnp.full((LANES,), bd2[0], jnp.float32)
  return decoder(hu, hm, edge_label_index[0], edge_label_index[1],
                 bd1[perm], Wd2.reshape(F)[perm], bd2p)
